# Initial kernel scaffold; baseline (speedup 1.0000x reference)
import jax
import jax.numpy as jnp
from jax.experimental import pallas as pl


def kernel(x, edge_index, W1, b1, W2, b2):
    raise NotImplementedError("write your pallas kernel here")


# speedup vs baseline: 26.2521x; 26.2521x over previous
import functools

import jax
import jax.numpy as jnp
from jax import lax
from jax.experimental import pallas as pl
from jax.experimental.pallas import tpu as pltpu
from jax.experimental.pallas import tpu_sc as plsc

N = 10000
D_IN = 128
D_HID = 16
N_CLASSES = 40
E = 320000

NS = 16
L = 16
NPAD = 10240
RPW = NPAD // NS
CH = 128
EPW = 20096
K = EPW // CH
EPAD = EPW * NS


def _mm1_body(x_ref, w_ref, o_ref):
    o_ref[...] = jnp.dot(x_ref[...], w_ref[...],
                         preferred_element_type=jnp.float32)


def _head_body(z_ref, w_ref, b_ref, o_ref):
    o = jnp.dot(z_ref[...], w_ref[...],
                preferred_element_type=jnp.float32) + b_ref[...]
    m = jnp.max(o, axis=1, keepdims=True)
    s = jnp.sum(jnp.exp(o - m), axis=1, keepdims=True)
    o_ref[...] = o - m - jnp.log(s)


def _rsqrt16(d):
    i = lax.bitcast_convert_type(d, jnp.int32)
    i = jnp.int32(0x5F3759DF) - lax.shift_right_logical(i, 1)
    y = lax.bitcast_convert_type(i, jnp.float32)
    for _ in range(4):
        y = y * (1.5 - 0.5 * d * y * y)
    return y


def _sc_body(h0, srcp, dstp, b1, z_out, hs1_t, hs2_t,
             deg_s, acc1_s, acc2_s,
             sidx, didx, rows, rbuf, dinv_t, degb, ones_t, b1_t, gsem):
    wid = lax.axis_index("s")
    rbase = wid * RPW
    rsl = pl.ds(rbase, RPW)

    pltpu.sync_copy(srcp.at[wid], sidx)
    pltpu.sync_copy(dstp.at[wid], didx)
    pltpu.sync_copy(b1, b1_t)

    def _fill_ones(j, _):
        ones_t[pl.ds(j * L, L)] = jnp.full((L,), 1.0, jnp.float32)
        return 0
    lax.fori_loop(0, CH // L, _fill_ones, 0)

    def _fill_deg(j, _):
        degb[pl.ds(j * L, L)] = jnp.full((L,), 1.0, jnp.float32)
        return 0
    lax.fori_loop(0, RPW // L, _fill_deg, 0)
    pltpu.sync_copy(degb, deg_s.at[rsl])
    plsc.subcore_barrier()

    def _deg_step(k, _):
        pltpu.sync_copy(ones_t, deg_s.at[didx.at[k]], add=True)
        return 0
    lax.fori_loop(0, K, _deg_step, 0)
    plsc.subcore_barrier()

    pltpu.sync_copy(deg_s.at[rsl], degb)

    def _rsq_step(j, _):
        sl = pl.ds(j * L, L)
        dinv_t[sl] = _rsqrt16(degb[sl])
        return 0
    lax.fori_loop(0, RPW // L, _rsq_step, 0)

    def _rowloop(fn):
        def _body(j, _):
            dv = dinv_t[pl.ds(j * L, L)]
            for t in range(L):
                fn(j * L + t, dv[t])
            return 0
        lax.fori_loop(0, RPW // L, _body, 0)

    pltpu.sync_copy(h0.at[rsl], rbuf)

    def _scale1(r, s):
        rbuf[r, :] = rbuf[r, :] * s
    _rowloop(_scale1)
    pltpu.sync_copy(rbuf, hs1_t.at[rsl])
    pltpu.sync_copy(rbuf, acc1_s.at[rsl])
    plsc.subcore_barrier()

    def _aggregate(table, acc):
        def _step(k, _):
            pltpu.async_copy(table.at[sidx.at[k]], rows, gsem).wait()
            pltpu.sync_copy(rows, acc.at[didx.at[k]], add=True)
            return 0
        lax.fori_loop(0, K, _step, 0)

    _aggregate(hs1_t, acc1_s)
    plsc.subcore_barrier()

    pltpu.sync_copy(acc1_s.at[rsl], rbuf)
    b1v = b1_t[...]

    def _mid(r, s):
        v = jnp.maximum(rbuf[r, :] * s + b1v, 0.0)
        rbuf[r, :] = v * s
    _rowloop(_mid)
    pltpu.sync_copy(rbuf, hs2_t.at[rsl])
    pltpu.sync_copy(rbuf, acc2_s.at[rsl])
    plsc.subcore_barrier()

    _aggregate(hs2_t, acc2_s)
    plsc.subcore_barrier()

    pltpu.sync_copy(acc2_s.at[rsl], rbuf)

    def _scale2(r, s):
        rbuf[r, :] = rbuf[r, :] * s
    _rowloop(_scale2)
    pltpu.sync_copy(rbuf, z_out.at[rsl])


_sc_agg = pl.kernel(
    _sc_body,
    out_type=(
        jax.ShapeDtypeStruct((NPAD, D_HID), jnp.float32),
        jax.ShapeDtypeStruct((NPAD, D_HID), jnp.float32),
        jax.ShapeDtypeStruct((NPAD, D_HID), jnp.float32),
    ),
    mesh=plsc.VectorSubcoreMesh(core_axis_name="c", subcore_axis_name="s",
                                num_cores=1),
    compiler_params=pltpu.CompilerParams(use_tc_tiling_on_sc=False),
    scratch_types=(
        pltpu.VMEM_SHARED((NPAD,), jnp.float32),
        pltpu.VMEM_SHARED((NPAD, D_HID), jnp.float32),
        pltpu.VMEM_SHARED((NPAD, D_HID), jnp.float32),
        pltpu.VMEM((K, CH), jnp.int32),
        pltpu.VMEM((K, CH), jnp.int32),
        pltpu.VMEM((CH, D_HID), jnp.float32),
        pltpu.VMEM((RPW, D_HID), jnp.float32),
        pltpu.VMEM((RPW,), jnp.float32),
        pltpu.VMEM((RPW,), jnp.float32),
        pltpu.VMEM((CH,), jnp.float32),
        pltpu.VMEM((D_HID,), jnp.float32),
        pltpu.SemaphoreType.DMA,
    ),
)


def kernel(x, edge_index, W1, b1, W2, b2):
    ei = edge_index.astype(jnp.int32)
    pad = jnp.full((EPAD - E,), N, jnp.int32)
    srcp = jnp.concatenate([ei[0], pad]).reshape(NS, K, CH)
    dstp = jnp.concatenate([ei[1], pad]).reshape(NS, K, CH)

    h0 = pl.pallas_call(
        _mm1_body,
        grid=(5,),
        in_specs=[
            pl.BlockSpec((2000, D_IN), lambda i: (i, 0)),
            pl.BlockSpec((D_IN, D_HID), lambda i: (0, 0)),
        ],
        out_specs=pl.BlockSpec((2000, D_HID), lambda i: (i, 0)),
        out_shape=jax.ShapeDtypeStruct((NPAD, D_HID), jnp.float32),
    )(x, W1)

    z, _, _ = _sc_agg(h0, srcp, dstp, b1)

    out = pl.pallas_call(
        _head_body,
        grid=(5,),
        in_specs=[
            pl.BlockSpec((2000, D_HID), lambda i: (i, 0)),
            pl.BlockSpec((D_HID, N_CLASSES), lambda i: (0, 0)),
            pl.BlockSpec((1, N_CLASSES), lambda i: (0, 0)),
        ],
        out_specs=pl.BlockSpec((2000, N_CLASSES), lambda i: (i, 0)),
        out_shape=jax.ShapeDtypeStruct((N, N_CLASSES), jnp.float32),
    )(z, W2, b2.reshape(1, N_CLASSES))
    return out


# speedup vs baseline: 35.6591x; 1.3583x over previous
import functools

import jax
import jax.numpy as jnp
from jax import lax
from jax.experimental import pallas as pl
from jax.experimental.pallas import tpu as pltpu
from jax.experimental.pallas import tpu_sc as plsc

N = 10000
D_IN = 128
D_HID = 16
N_CLASSES = 40
E = 320000

NS = 16
L = 16
NPAD = 10240
RPW = NPAD // NS
CH = 128
EPW = 20480
K = EPW // CH
EPAD = EPW * NS


def _mm1_body(x_ref, w_ref, o_ref):
    o_ref[...] = jnp.dot(x_ref[...], w_ref[...],
                         preferred_element_type=jnp.float32)


def _head_body(z_ref, w_ref, b_ref, o_ref):
    o = jnp.dot(z_ref[...], w_ref[...],
                preferred_element_type=jnp.float32) + b_ref[...]
    m = jnp.max(o, axis=1, keepdims=True)
    s = jnp.sum(jnp.exp(o - m), axis=1, keepdims=True)
    o_ref[...] = o - m - jnp.log(s)


def _rsqrt16(d):
    i = lax.bitcast_convert_type(d, jnp.int32)
    i = jnp.int32(0x5F3759DF) - lax.shift_right_logical(i, 1)
    y = lax.bitcast_convert_type(i, jnp.float32)
    for _ in range(4):
        y = y * (1.5 - 0.5 * d * y * y)
    return y


def _sc_body(h0, srcp, dstp, b1, z_out, hs1_t, hs2_t,
             deg_s, acc1_s, acc2_s,
             sidx, didx, rows, rbuf, dinv_t, degb, ones_t, b1_t,
             gsem, ssem, dsem, hsem):
    wid = lax.axis_index("s")
    rbase = wid * RPW
    rsl = pl.ds(rbase, RPW)

    pltpu.sync_copy(srcp.at[wid], sidx)
    pltpu.sync_copy(dstp.at[wid], didx)
    pltpu.sync_copy(b1, b1_t)
    pltpu.async_copy(h0.at[rsl], rbuf, hsem)

    def _fill_ones(j, _):
        ones_t[pl.ds(j * L, L)] = jnp.full((L,), 1.0, jnp.float32)
        return 0
    lax.fori_loop(0, CH // L, _fill_ones, 0)

    def _fill_deg(j, _):
        degb[pl.ds(j * L, L)] = jnp.full((L,), 1.0, jnp.float32)
        return 0
    lax.fori_loop(0, RPW // L, _fill_deg, 0)
    pltpu.sync_copy(degb, deg_s.at[rsl])
    plsc.subcore_barrier()

    def _deg_wait():
        pltpu.make_async_copy(ones_t, deg_s.at[didx.at[0]], dsem).wait()

    def _deg_step(k, _):
        pltpu.async_copy(ones_t, deg_s.at[didx.at[k]], dsem, add=True)

        @pl.when(k >= 4)
        def _():
            _deg_wait()
        return 0
    lax.fori_loop(0, K, _deg_step, 0)
    for _ in range(4):
        _deg_wait()
    plsc.subcore_barrier()

    pltpu.sync_copy(deg_s.at[rsl], degb)

    def _rsq_step(j, _):
        sl = pl.ds(j * L, L)
        dinv_t[sl] = _rsqrt16(degb[sl])
        return 0
    lax.fori_loop(0, RPW // L, _rsq_step, 0)

    def _rowloop(fn):
        def _body(j, _):
            dv = dinv_t[pl.ds(j * L, L)]
            for t in range(L):
                fn(j * L + t, dv[t])
            return 0
        lax.fori_loop(0, RPW // L, _body, 0)

    pltpu.make_async_copy(h0.at[rsl], rbuf, hsem).wait()

    def _scale1(r, s):
        rbuf[r, :] = rbuf[r, :] * s
    _rowloop(_scale1)
    pltpu.sync_copy(rbuf, hs1_t.at[rsl])
    pltpu.sync_copy(rbuf, acc1_s.at[rsl])
    plsc.subcore_barrier()

    def _aggregate(table, acc):
        def _start_g(k, b):
            pltpu.async_copy(table.at[sidx.at[k]], rows.at[b], gsem.at[b])

        def _wait_g(b):
            pltpu.make_async_copy(table.at[sidx.at[0]], rows.at[b],
                                  gsem.at[b]).wait()

        def _start_s(k, b):
            pltpu.async_copy(rows.at[b], acc.at[didx.at[k]], ssem.at[b],
                             add=True)

        def _wait_s(b):
            pltpu.make_async_copy(rows.at[b], acc.at[didx.at[0]],
                                  ssem.at[b]).wait()

        _start_g(0, 0)
        _start_g(1, 1)

        def _grp(g, _):
            for t in range(4):
                k = g * 4 + t
                _wait_g(t)
                _start_s(k, t)
                bn = (t + 2) % 4

                @pl.when(k + 2 < K)
                def _():
                    @pl.when(k >= 2)
                    def _():
                        _wait_s(bn)
                    _start_g(k + 2, bn)
            return 0
        lax.fori_loop(0, K // 4, _grp, 0)
        for b in range(4):
            _wait_s(b)

    _aggregate(hs1_t, acc1_s)
    plsc.subcore_barrier()

    pltpu.sync_copy(acc1_s.at[rsl], rbuf)
    b1v = b1_t[...]

    def _mid(r, s):
        v = jnp.maximum(rbuf[r, :] * s + b1v, 0.0)
        rbuf[r, :] = v * s
    _rowloop(_mid)
    pltpu.sync_copy(rbuf, hs2_t.at[rsl])
    pltpu.sync_copy(rbuf, acc2_s.at[rsl])
    plsc.subcore_barrier()

    _aggregate(hs2_t, acc2_s)
    plsc.subcore_barrier()

    pltpu.sync_copy(acc2_s.at[rsl], rbuf)

    def _scale2(r, s):
        rbuf[r, :] = rbuf[r, :] * s
    _rowloop(_scale2)
    pltpu.sync_copy(rbuf, z_out.at[rsl])


_sc_agg = pl.kernel(
    _sc_body,
    out_type=(
        jax.ShapeDtypeStruct((NPAD, D_HID), jnp.float32),
        jax.ShapeDtypeStruct((NPAD, D_HID), jnp.float32),
        jax.ShapeDtypeStruct((NPAD, D_HID), jnp.float32),
    ),
    mesh=plsc.VectorSubcoreMesh(core_axis_name="c", subcore_axis_name="s",
                                num_cores=1),
    compiler_params=pltpu.CompilerParams(use_tc_tiling_on_sc=False),
    scratch_types=(
        pltpu.VMEM_SHARED((NPAD,), jnp.float32),
        pltpu.VMEM_SHARED((NPAD, D_HID), jnp.float32),
        pltpu.VMEM_SHARED((NPAD, D_HID), jnp.float32),
        pltpu.VMEM((K, CH), jnp.int32),
        pltpu.VMEM((K, CH), jnp.int32),
        pltpu.VMEM((4, CH, D_HID), jnp.float32),
        pltpu.VMEM((RPW, D_HID), jnp.float32),
        pltpu.VMEM((RPW,), jnp.float32),
        pltpu.VMEM((RPW,), jnp.float32),
        pltpu.VMEM((CH,), jnp.float32),
        pltpu.VMEM((D_HID,), jnp.float32),
        pltpu.SemaphoreType.DMA((4,)),
        pltpu.SemaphoreType.DMA((4,)),
        pltpu.SemaphoreType.DMA,
        pltpu.SemaphoreType.DMA,
    ),
)


def kernel(x, edge_index, W1, b1, W2, b2):
    ei = edge_index.astype(jnp.int32)
    pad = jnp.full((EPAD - E,), N, jnp.int32)
    srcp = jnp.concatenate([ei[0], pad]).reshape(NS, K, CH)
    dstp = jnp.concatenate([ei[1], pad]).reshape(NS, K, CH)

    h0 = pl.pallas_call(
        _mm1_body,
        grid=(5,),
        in_specs=[
            pl.BlockSpec((2000, D_IN), lambda i: (i, 0)),
            pl.BlockSpec((D_IN, D_HID), lambda i: (0, 0)),
        ],
        out_specs=pl.BlockSpec((2000, D_HID), lambda i: (i, 0)),
        out_shape=jax.ShapeDtypeStruct((NPAD, D_HID), jnp.float32),
    )(x, W1)

    z, _, _ = _sc_agg(h0, srcp, dstp, b1)

    out = pl.pallas_call(
        _head_body,
        grid=(5,),
        in_specs=[
            pl.BlockSpec((2000, D_HID), lambda i: (i, 0)),
            pl.BlockSpec((D_HID, N_CLASSES), lambda i: (0, 0)),
            pl.BlockSpec((1, N_CLASSES), lambda i: (0, 0)),
        ],
        out_specs=pl.BlockSpec((2000, N_CLASSES), lambda i: (i, 0)),
        out_shape=jax.ShapeDtypeStruct((N, N_CLASSES), jnp.float32),
    )(z, W2, b2.reshape(1, N_CLASSES))
    return out


# speedup vs baseline: 39.4428x; 1.1061x over previous
import functools

import jax
import jax.numpy as jnp
from jax import lax
from jax.experimental import pallas as pl
from jax.experimental.pallas import tpu as pltpu
from jax.experimental.pallas import tpu_sc as plsc

N = 10000
D_IN = 128
D_HID = 16
N_CLASSES = 40
E = 320000

NS = 16
L = 16
NPAD = 10240
RPW = NPAD // NS
CH = 256
EPW = 20480
K = EPW // CH
EPAD = EPW * NS


def _mm1_body(x_ref, w_ref, o_ref):
    o_ref[...] = jnp.dot(x_ref[...], w_ref[...],
                         preferred_element_type=jnp.float32)


def _head_body(z_ref, w_ref, b_ref, o_ref):
    o = jnp.dot(z_ref[...], w_ref[...],
                preferred_element_type=jnp.float32) + b_ref[...]
    m = jnp.max(o, axis=1, keepdims=True)
    s = jnp.sum(jnp.exp(o - m), axis=1, keepdims=True)
    o_ref[...] = o - m - jnp.log(s)


def _rsqrt16(d):
    i = lax.bitcast_convert_type(d, jnp.int32)
    i = jnp.int32(0x5F3759DF) - lax.shift_right_logical(i, 1)
    y = lax.bitcast_convert_type(i, jnp.float32)
    for _ in range(4):
        y = y * (1.5 - 0.5 * d * y * y)
    return y


def _sc_body(h0, srcp, dstp, b1, z_out, hs1_t, hs2_t,
             deg_s, acc1_s, acc2_s,
             sidx, didx, rows, rbuf, dinv_t, degb, ones_t, b1_t,
             gsem, ssem, dsem, hsem):
    wid = lax.axis_index("s")
    rbase = wid * RPW
    rsl = pl.ds(rbase, RPW)

    pltpu.sync_copy(srcp.at[wid], sidx)
    pltpu.sync_copy(dstp.at[wid], didx)
    pltpu.sync_copy(b1, b1_t)
    pltpu.async_copy(h0.at[rsl], rbuf, hsem)

    def _fill_ones(j, _):
        ones_t[pl.ds(j * L, L)] = jnp.full((L,), 1.0, jnp.float32)
        return 0
    lax.fori_loop(0, CH // L, _fill_ones, 0)

    def _fill_deg(j, _):
        degb[pl.ds(j * L, L)] = jnp.full((L,), 1.0, jnp.float32)
        return 0
    lax.fori_loop(0, RPW // L, _fill_deg, 0)
    pltpu.sync_copy(degb, deg_s.at[rsl])
    plsc.subcore_barrier()

    def _deg_wait():
        pltpu.make_async_copy(ones_t, deg_s.at[didx.at[0]], dsem).wait()

    def _deg_step(k, _):
        pltpu.async_copy(ones_t, deg_s.at[didx.at[k]], dsem, add=True)

        @pl.when(k >= 4)
        def _():
            _deg_wait()
        return 0
    lax.fori_loop(0, K, _deg_step, 0)
    for _ in range(4):
        _deg_wait()
    plsc.subcore_barrier()

    pltpu.sync_copy(deg_s.at[rsl], degb)

    def _rsq_step(j, _):
        sl = pl.ds(j * L, L)
        dinv_t[sl] = _rsqrt16(degb[sl])
        return 0
    lax.fori_loop(0, RPW // L, _rsq_step, 0)

    def _rowloop(fn):
        def _body(j, _):
            dv = dinv_t[pl.ds(j * L, L)]
            for t in range(L):
                fn(j * L + t, dv[t])
            return 0
        lax.fori_loop(0, RPW // L, _body, 0)

    pltpu.make_async_copy(h0.at[rsl], rbuf, hsem).wait()

    def _scale1(r, s):
        rbuf[r, :] = rbuf[r, :] * s
    _rowloop(_scale1)
    pltpu.sync_copy(rbuf, hs1_t.at[rsl])
    pltpu.sync_copy(rbuf, acc1_s.at[rsl])
    plsc.subcore_barrier()

    def _aggregate(table, acc):
        def _start_g(k, b):
            pltpu.async_copy(table.at[sidx.at[k]], rows.at[b], gsem.at[b])

        def _wait_g(b):
            pltpu.make_async_copy(table.at[sidx.at[0]], rows.at[b],
                                  gsem.at[b]).wait()

        def _start_s(k, b):
            pltpu.async_copy(rows.at[b], acc.at[didx.at[k]], ssem.at[b],
                             add=True)

        def _wait_s(b):
            pltpu.make_async_copy(rows.at[b], acc.at[didx.at[0]],
                                  ssem.at[b]).wait()

        _start_g(0, 0)
        _start_g(1, 1)

        def _grp(g, _):
            for t in range(4):
                k = g * 4 + t
                _wait_g(t)
                _start_s(k, t)
                bn = (t + 2) % 4

                @pl.when(k + 2 < K)
                def _():
                    @pl.when(k >= 2)
                    def _():
                        _wait_s(bn)
                    _start_g(k + 2, bn)
            return 0
        lax.fori_loop(0, K // 4, _grp, 0)
        for b in range(4):
            _wait_s(b)

    _aggregate(hs1_t, acc1_s)
    plsc.subcore_barrier()

    pltpu.sync_copy(acc1_s.at[rsl], rbuf)
    b1v = b1_t[...]

    def _mid(r, s):
        v = jnp.maximum(rbuf[r, :] * s + b1v, 0.0)
        rbuf[r, :] = v * s
    _rowloop(_mid)
    pltpu.sync_copy(rbuf, hs2_t.at[rsl])
    pltpu.sync_copy(rbuf, acc2_s.at[rsl])
    plsc.subcore_barrier()

    _aggregate(hs2_t, acc2_s)
    plsc.subcore_barrier()

    pltpu.sync_copy(acc2_s.at[rsl], rbuf)

    def _scale2(r, s):
        rbuf[r, :] = rbuf[r, :] * s
    _rowloop(_scale2)
    pltpu.sync_copy(rbuf, z_out.at[rsl])


_sc_agg = pl.kernel(
    _sc_body,
    out_type=(
        jax.ShapeDtypeStruct((NPAD, D_HID), jnp.float32),
        jax.ShapeDtypeStruct((NPAD, D_HID), jnp.float32),
        jax.ShapeDtypeStruct((NPAD, D_HID), jnp.float32),
    ),
    mesh=plsc.VectorSubcoreMesh(core_axis_name="c", subcore_axis_name="s",
                                num_cores=1),
    compiler_params=pltpu.CompilerParams(use_tc_tiling_on_sc=False),
    scratch_types=(
        pltpu.VMEM_SHARED((NPAD,), jnp.float32),
        pltpu.VMEM_SHARED((NPAD, D_HID), jnp.float32),
        pltpu.VMEM_SHARED((NPAD, D_HID), jnp.float32),
        pltpu.VMEM((K, CH), jnp.int32),
        pltpu.VMEM((K, CH), jnp.int32),
        pltpu.VMEM((4, CH, D_HID), jnp.float32),
        pltpu.VMEM((RPW, D_HID), jnp.float32),
        pltpu.VMEM((RPW,), jnp.float32),
        pltpu.VMEM((RPW,), jnp.float32),
        pltpu.VMEM((CH,), jnp.float32),
        pltpu.VMEM((D_HID,), jnp.float32),
        pltpu.SemaphoreType.DMA((4,)),
        pltpu.SemaphoreType.DMA((4,)),
        pltpu.SemaphoreType.DMA,
        pltpu.SemaphoreType.DMA,
    ),
)


def kernel(x, edge_index, W1, b1, W2, b2):
    ei = edge_index.astype(jnp.int32)
    pad = jnp.full((EPAD - E,), N, jnp.int32)
    srcp = jnp.concatenate([ei[0], pad]).reshape(NS, K, CH)
    dstp = jnp.concatenate([ei[1], pad]).reshape(NS, K, CH)

    h0 = pl.pallas_call(
        _mm1_body,
        grid=(5,),
        in_specs=[
            pl.BlockSpec((2000, D_IN), lambda i: (i, 0)),
            pl.BlockSpec((D_IN, D_HID), lambda i: (0, 0)),
        ],
        out_specs=pl.BlockSpec((2000, D_HID), lambda i: (i, 0)),
        out_shape=jax.ShapeDtypeStruct((NPAD, D_HID), jnp.float32),
    )(x, W1)

    z, _, _ = _sc_agg(h0, srcp, dstp, b1)

    out = pl.pallas_call(
        _head_body,
        grid=(5,),
        in_specs=[
            pl.BlockSpec((2000, D_HID), lambda i: (i, 0)),
            pl.BlockSpec((D_HID, N_CLASSES), lambda i: (0, 0)),
            pl.BlockSpec((1, N_CLASSES), lambda i: (0, 0)),
        ],
        out_specs=pl.BlockSpec((2000, N_CLASSES), lambda i: (i, 0)),
        out_shape=jax.ShapeDtypeStruct((N, N_CLASSES), jnp.float32),
    )(z, W2, b2.reshape(1, N_CLASSES))
    return out


# speedup vs baseline: 41.4677x; 1.0513x over previous
import functools

import jax
import jax.numpy as jnp
from jax import lax
from jax.experimental import pallas as pl
from jax.experimental.pallas import tpu as pltpu
from jax.experimental.pallas import tpu_sc as plsc

N = 10000
D_IN = 128
D_HID = 16
N_CLASSES = 40
E = 320000

NS = 16
L = 16
NPAD = 10240
RPW = NPAD // NS
CH = 512
EPW = 20480
K = EPW // CH
EPAD = EPW * NS


def _mm1_body(x_ref, w_ref, o_ref):
    o_ref[...] = jnp.dot(x_ref[...], w_ref[...],
                         preferred_element_type=jnp.float32)


def _head_body(z_ref, w_ref, b_ref, o_ref):
    o = jnp.dot(z_ref[...], w_ref[...],
                preferred_element_type=jnp.float32) + b_ref[...]
    m = jnp.max(o, axis=1, keepdims=True)
    s = jnp.sum(jnp.exp(o - m), axis=1, keepdims=True)
    o_ref[...] = o - m - jnp.log(s)


def _rsqrt16(d):
    i = lax.bitcast_convert_type(d, jnp.int32)
    i = jnp.int32(0x5F3759DF) - lax.shift_right_logical(i, 1)
    y = lax.bitcast_convert_type(i, jnp.float32)
    for _ in range(4):
        y = y * (1.5 - 0.5 * d * y * y)
    return y


def _sc_body(h0, srcp, dstp, b1, z_out, hs1_t, hs2_t,
             deg_s, acc1_s, acc2_s,
             sidx, didx, rows, rbuf, dinv_t, degb, ones_t, b1_t,
             gsem, ssem, dsem, hsem):
    wid = lax.axis_index("s")
    rbase = wid * RPW
    rsl = pl.ds(rbase, RPW)

    pltpu.sync_copy(srcp.at[wid], sidx)
    pltpu.sync_copy(dstp.at[wid], didx)
    pltpu.sync_copy(b1, b1_t)
    pltpu.async_copy(h0.at[rsl], rbuf, hsem)

    def _fill_ones(j, _):
        ones_t[pl.ds(j * L, L)] = jnp.full((L,), 1.0, jnp.float32)
        return 0
    lax.fori_loop(0, CH // L, _fill_ones, 0)

    def _fill_deg(j, _):
        degb[pl.ds(j * L, L)] = jnp.full((L,), 1.0, jnp.float32)
        return 0
    lax.fori_loop(0, RPW // L, _fill_deg, 0)
    pltpu.sync_copy(degb, deg_s.at[rsl])
    plsc.subcore_barrier()

    def _deg_wait():
        pltpu.make_async_copy(ones_t, deg_s.at[didx.at[0]], dsem).wait()

    def _deg_step(k, _):
        pltpu.async_copy(ones_t, deg_s.at[didx.at[k]], dsem, add=True)

        @pl.when(k >= 4)
        def _():
            _deg_wait()
        return 0
    lax.fori_loop(0, K, _deg_step, 0)
    for _ in range(4):
        _deg_wait()
    plsc.subcore_barrier()

    pltpu.sync_copy(deg_s.at[rsl], degb)

    def _rsq_step(j, _):
        sl = pl.ds(j * L, L)
        dinv_t[sl] = _rsqrt16(degb[sl])
        return 0
    lax.fori_loop(0, RPW // L, _rsq_step, 0)

    def _rowloop(fn):
        def _body(j, _):
            dv = dinv_t[pl.ds(j * L, L)]
            for t in range(L):
                fn(j * L + t, dv[t])
            return 0
        lax.fori_loop(0, RPW // L, _body, 0)

    pltpu.make_async_copy(h0.at[rsl], rbuf, hsem).wait()

    def _scale1(r, s):
        rbuf[r, :] = rbuf[r, :] * s
    _rowloop(_scale1)
    pltpu.sync_copy(rbuf, hs1_t.at[rsl])
    pltpu.sync_copy(rbuf, acc1_s.at[rsl])
    plsc.subcore_barrier()

    def _aggregate(table, acc):
        def _start_g(k, b):
            pltpu.async_copy(table.at[sidx.at[k]], rows.at[b], gsem.at[b])

        def _wait_g(b):
            pltpu.make_async_copy(table.at[sidx.at[0]], rows.at[b],
                                  gsem.at[b]).wait()

        def _start_s(k, b):
            pltpu.async_copy(rows.at[b], acc.at[didx.at[k]], ssem.at[b],
                             add=True)

        def _wait_s(b):
            pltpu.make_async_copy(rows.at[b], acc.at[didx.at[0]],
                                  ssem.at[b]).wait()

        _start_g(0, 0)
        _start_g(1, 1)

        def _grp(g, _):
            for t in range(4):
                k = g * 4 + t
                _wait_g(t)
                _start_s(k, t)
                bn = (t + 2) % 4

                @pl.when(k + 2 < K)
                def _():
                    @pl.when(k >= 2)
                    def _():
                        _wait_s(bn)
                    _start_g(k + 2, bn)
            return 0
        lax.fori_loop(0, K // 4, _grp, 0)
        for b in range(4):
            _wait_s(b)

    _aggregate(hs1_t, acc1_s)
    plsc.subcore_barrier()

    pltpu.sync_copy(acc1_s.at[rsl], rbuf)
    b1v = b1_t[...]

    def _mid(r, s):
        v = jnp.maximum(rbuf[r, :] * s + b1v, 0.0)
        rbuf[r, :] = v * s
    _rowloop(_mid)
    pltpu.sync_copy(rbuf, hs2_t.at[rsl])
    pltpu.sync_copy(rbuf, acc2_s.at[rsl])
    plsc.subcore_barrier()

    _aggregate(hs2_t, acc2_s)
    plsc.subcore_barrier()

    pltpu.sync_copy(acc2_s.at[rsl], rbuf)

    def _scale2(r, s):
        rbuf[r, :] = rbuf[r, :] * s
    _rowloop(_scale2)
    pltpu.sync_copy(rbuf, z_out.at[rsl])


_sc_agg = pl.kernel(
    _sc_body,
    out_type=(
        jax.ShapeDtypeStruct((NPAD, D_HID), jnp.float32),
        jax.ShapeDtypeStruct((NPAD, D_HID), jnp.float32),
        jax.ShapeDtypeStruct((NPAD, D_HID), jnp.float32),
    ),
    mesh=plsc.VectorSubcoreMesh(core_axis_name="c", subcore_axis_name="s",
                                num_cores=1),
    compiler_params=pltpu.CompilerParams(use_tc_tiling_on_sc=False),
    scratch_types=(
        pltpu.VMEM_SHARED((NPAD,), jnp.float32),
        pltpu.VMEM_SHARED((NPAD, D_HID), jnp.float32),
        pltpu.VMEM_SHARED((NPAD, D_HID), jnp.float32),
        pltpu.VMEM((K, CH), jnp.int32),
        pltpu.VMEM((K, CH), jnp.int32),
        pltpu.VMEM((4, CH, D_HID), jnp.float32),
        pltpu.VMEM((RPW, D_HID), jnp.float32),
        pltpu.VMEM((RPW,), jnp.float32),
        pltpu.VMEM((RPW,), jnp.float32),
        pltpu.VMEM((CH,), jnp.float32),
        pltpu.VMEM((D_HID,), jnp.float32),
        pltpu.SemaphoreType.DMA((4,)),
        pltpu.SemaphoreType.DMA((4,)),
        pltpu.SemaphoreType.DMA,
        pltpu.SemaphoreType.DMA,
    ),
)


def kernel(x, edge_index, W1, b1, W2, b2):
    ei = edge_index.astype(jnp.int32)
    pad = jnp.full((EPAD - E,), N, jnp.int32)
    srcp = jnp.concatenate([ei[0], pad]).reshape(NS, K, CH)
    dstp = jnp.concatenate([ei[1], pad]).reshape(NS, K, CH)

    h0 = pl.pallas_call(
        _mm1_body,
        grid=(5,),
        in_specs=[
            pl.BlockSpec((2000, D_IN), lambda i: (i, 0)),
            pl.BlockSpec((D_IN, D_HID), lambda i: (0, 0)),
        ],
        out_specs=pl.BlockSpec((2000, D_HID), lambda i: (i, 0)),
        out_shape=jax.ShapeDtypeStruct((NPAD, D_HID), jnp.float32),
    )(x, W1)

    z, _, _ = _sc_agg(h0, srcp, dstp, b1)

    out = pl.pallas_call(
        _head_body,
        grid=(5,),
        in_specs=[
            pl.BlockSpec((2000, D_HID), lambda i: (i, 0)),
            pl.BlockSpec((D_HID, N_CLASSES), lambda i: (0, 0)),
            pl.BlockSpec((1, N_CLASSES), lambda i: (0, 0)),
        ],
        out_specs=pl.BlockSpec((2000, N_CLASSES), lambda i: (i, 0)),
        out_shape=jax.ShapeDtypeStruct((N, N_CLASSES), jnp.float32),
    )(z, W2, b2.reshape(1, N_CLASSES))
    return out


# speedup vs baseline: 42.3408x; 1.0211x over previous
import functools

import jax
import jax.numpy as jnp
from jax import lax
from jax.experimental import pallas as pl
from jax.experimental.pallas import tpu as pltpu
from jax.experimental.pallas import tpu_sc as plsc

N = 10000
D_IN = 128
D_HID = 16
N_CLASSES = 40
E = 320000

NS = 16
L = 16
NPAD = 10240
RPW = NPAD // NS
CH = 256
EPW = 20480
K = EPW // CH
NSLOT = 8
EPAD = EPW * NS


def _mm1_body(x_ref, w_ref, o_ref):
    o_ref[...] = jnp.dot(x_ref[...], w_ref[...],
                         preferred_element_type=jnp.float32)


def _head_body(z_ref, w_ref, b_ref, o_ref):
    o = jnp.dot(z_ref[...], w_ref[...],
                preferred_element_type=jnp.float32) + b_ref[...]
    m = jnp.max(o, axis=1, keepdims=True)
    s = jnp.sum(jnp.exp(o - m), axis=1, keepdims=True)
    o_ref[...] = o - m - jnp.log(s)


def _rsqrt16(d):
    i = lax.bitcast_convert_type(d, jnp.int32)
    i = jnp.int32(0x5F3759DF) - lax.shift_right_logical(i, 1)
    y = lax.bitcast_convert_type(i, jnp.float32)
    for _ in range(4):
        y = y * (1.5 - 0.5 * d * y * y)
    return y


def _sc_body(h0, srcp, dstp, b1, z_out, hs1_t, hs2_t,
             deg_s, acc1_s, acc2_s,
             sidx, didx, rows, rbuf, dinv_t, degb, ones_t, b1_t,
             gsem, ssem, dsem, hsem):
    wid = lax.axis_index("s")
    rbase = wid * RPW
    rsl = pl.ds(rbase, RPW)

    pltpu.sync_copy(srcp.at[wid], sidx)
    pltpu.sync_copy(dstp.at[wid], didx)
    pltpu.sync_copy(b1, b1_t)
    pltpu.async_copy(h0.at[rsl], rbuf, hsem)

    def _fill_ones(j, _):
        ones_t[pl.ds(j * L, L)] = jnp.full((L,), 1.0, jnp.float32)
        return 0
    lax.fori_loop(0, CH // L, _fill_ones, 0)

    def _fill_deg(j, _):
        degb[pl.ds(j * L, L)] = jnp.full((L,), 1.0, jnp.float32)
        return 0
    lax.fori_loop(0, RPW // L, _fill_deg, 0)
    pltpu.sync_copy(degb, deg_s.at[rsl])
    plsc.subcore_barrier()

    def _deg_wait():
        pltpu.make_async_copy(ones_t, deg_s.at[didx.at[0]], dsem).wait()

    def _deg_step(k, _):
        pltpu.async_copy(ones_t, deg_s.at[didx.at[k]], dsem, add=True)

        @pl.when(k >= 4)
        def _():
            _deg_wait()
        return 0
    lax.fori_loop(0, K, _deg_step, 0)
    for _ in range(4):
        _deg_wait()
    plsc.subcore_barrier()

    pltpu.sync_copy(deg_s.at[rsl], degb)

    def _rsq_step(j, _):
        sl = pl.ds(j * L, L)
        dinv_t[sl] = _rsqrt16(degb[sl])
        return 0
    lax.fori_loop(0, RPW // L, _rsq_step, 0)

    def _rowloop(fn):
        def _body(j, _):
            dv = dinv_t[pl.ds(j * L, L)]
            for t in range(L):
                fn(j * L + t, dv[t])
            return 0
        lax.fori_loop(0, RPW // L, _body, 0)

    pltpu.make_async_copy(h0.at[rsl], rbuf, hsem).wait()

    def _scale1(r, s):
        rbuf[r, :] = rbuf[r, :] * s
    _rowloop(_scale1)
    pltpu.sync_copy(rbuf, hs1_t.at[rsl])
    pltpu.sync_copy(rbuf, acc1_s.at[rsl])
    plsc.subcore_barrier()

    def _aggregate(table, acc):
        def _start_g(k, b):
            pltpu.async_copy(table.at[sidx.at[k]], rows.at[b], gsem.at[b])

        def _wait_g(b):
            pltpu.make_async_copy(table.at[sidx.at[0]], rows.at[b],
                                  gsem.at[b]).wait()

        def _start_s(k, b):
            pltpu.async_copy(rows.at[b], acc.at[didx.at[k]], ssem.at[b],
                             add=True)

        def _wait_s(b):
            pltpu.make_async_copy(rows.at[b], acc.at[didx.at[0]],
                                  ssem.at[b]).wait()

        half = NSLOT // 2
        for b in range(half):
            _start_g(b, b)

        def _grp(g, _):
            for t in range(NSLOT):
                k = g * NSLOT + t
                _wait_g(t)
                _start_s(k, t)
                bn = (t + half) % NSLOT

                @pl.when(k + half < K)
                def _():
                    @pl.when(k >= half)
                    def _():
                        _wait_s(bn)
                    _start_g(k + half, bn)
            return 0
        lax.fori_loop(0, K // NSLOT, _grp, 0)
        for b in range(NSLOT):
            _wait_s(b)

    _aggregate(hs1_t, acc1_s)
    plsc.subcore_barrier()

    pltpu.sync_copy(acc1_s.at[rsl], rbuf)
    b1v = b1_t[...]

    def _mid(r, s):
        v = jnp.maximum(rbuf[r, :] * s + b1v, 0.0)
        rbuf[r, :] = v * s
    _rowloop(_mid)
    pltpu.sync_copy(rbuf, hs2_t.at[rsl])
    pltpu.sync_copy(rbuf, acc2_s.at[rsl])
    plsc.subcore_barrier()

    _aggregate(hs2_t, acc2_s)
    plsc.subcore_barrier()

    pltpu.sync_copy(acc2_s.at[rsl], rbuf)

    def _scale2(r, s):
        rbuf[r, :] = rbuf[r, :] * s
    _rowloop(_scale2)
    pltpu.sync_copy(rbuf, z_out.at[rsl])


_sc_agg = pl.kernel(
    _sc_body,
    out_type=(
        jax.ShapeDtypeStruct((NPAD, D_HID), jnp.float32),
        jax.ShapeDtypeStruct((NPAD, D_HID), jnp.float32),
        jax.ShapeDtypeStruct((NPAD, D_HID), jnp.float32),
    ),
    mesh=plsc.VectorSubcoreMesh(core_axis_name="c", subcore_axis_name="s",
                                num_cores=1),
    compiler_params=pltpu.CompilerParams(use_tc_tiling_on_sc=False),
    scratch_types=(
        pltpu.VMEM_SHARED((NPAD,), jnp.float32),
        pltpu.VMEM_SHARED((NPAD, D_HID), jnp.float32),
        pltpu.VMEM_SHARED((NPAD, D_HID), jnp.float32),
        pltpu.VMEM((K, CH), jnp.int32),
        pltpu.VMEM((K, CH), jnp.int32),
        pltpu.VMEM((NSLOT, CH, D_HID), jnp.float32),
        pltpu.VMEM((RPW, D_HID), jnp.float32),
        pltpu.VMEM((RPW,), jnp.float32),
        pltpu.VMEM((RPW,), jnp.float32),
        pltpu.VMEM((CH,), jnp.float32),
        pltpu.VMEM((D_HID,), jnp.float32),
        pltpu.SemaphoreType.DMA((NSLOT,)),
        pltpu.SemaphoreType.DMA((NSLOT,)),
        pltpu.SemaphoreType.DMA,
        pltpu.SemaphoreType.DMA,
    ),
)


def kernel(x, edge_index, W1, b1, W2, b2):
    ei = edge_index.astype(jnp.int32)
    pad = jnp.full((EPAD - E,), N, jnp.int32)
    srcp = jnp.concatenate([ei[0], pad]).reshape(NS, K, CH)
    dstp = jnp.concatenate([ei[1], pad]).reshape(NS, K, CH)

    h0 = pl.pallas_call(
        _mm1_body,
        grid=(5,),
        in_specs=[
            pl.BlockSpec((2000, D_IN), lambda i: (i, 0)),
            pl.BlockSpec((D_IN, D_HID), lambda i: (0, 0)),
        ],
        out_specs=pl.BlockSpec((2000, D_HID), lambda i: (i, 0)),
        out_shape=jax.ShapeDtypeStruct((NPAD, D_HID), jnp.float32),
    )(x, W1)

    z, _, _ = _sc_agg(h0, srcp, dstp, b1)

    out = pl.pallas_call(
        _head_body,
        grid=(5,),
        in_specs=[
            pl.BlockSpec((2000, D_HID), lambda i: (i, 0)),
            pl.BlockSpec((D_HID, N_CLASSES), lambda i: (0, 0)),
            pl.BlockSpec((1, N_CLASSES), lambda i: (0, 0)),
        ],
        out_specs=pl.BlockSpec((2000, N_CLASSES), lambda i: (i, 0)),
        out_shape=jax.ShapeDtypeStruct((N, N_CLASSES), jnp.float32),
    )(z, W2, b2.reshape(1, N_CLASSES))
    return out


# speedup vs baseline: 53.7374x; 1.2692x over previous
import functools

import jax
import jax.numpy as jnp
from jax import lax
from jax.experimental import pallas as pl
from jax.experimental.pallas import tpu as pltpu
from jax.experimental.pallas import tpu_sc as plsc

N = 10000
D_IN = 128
D_HID = 16
N_CLASSES = 40
E = 320000

NS = 16
L = 16
NPAD = 10240
RPW = NPAD // NS
CH = 256
EPW = 20480
K = EPW // CH
NSLOT = 8
EPAD = EPW * NS


def _mm1_body(x_ref, w_ref, o_ref):
    o_ref[...] = jnp.dot(x_ref[...], w_ref[...],
                         preferred_element_type=jnp.float32)


def _head_body(z_ref, w_ref, b_ref, o_ref):
    o = jnp.dot(z_ref[...], w_ref[...],
                preferred_element_type=jnp.float32) + b_ref[...]
    m = jnp.max(o, axis=1, keepdims=True)
    s = jnp.sum(jnp.exp(o - m), axis=1, keepdims=True)
    o_ref[...] = o - m - jnp.log(s)


def _rsqrt16(d):
    i = lax.bitcast_convert_type(d, jnp.int32)
    i = jnp.int32(0x5F3759DF) - lax.shift_right_logical(i, 1)
    y = lax.bitcast_convert_type(i, jnp.float32)
    for _ in range(4):
        y = y * (1.5 - 0.5 * d * y * y)
    return y


def _sc_body(h0, srcp, dstp, b1, z_out, hs1_t, hs2_t,
             deg_s, acc1_s, acc2_s, hs1_s, hs2_s,
             sidx, didx, rows, rbuf, dinv_t, degb, ones_t, b1_t,
             gsem, ssem, dsem, hsem):
    wid = lax.axis_index("s")
    rbase = wid * RPW
    rsl = pl.ds(rbase, RPW)

    pltpu.sync_copy(srcp.at[wid], sidx)
    pltpu.sync_copy(dstp.at[wid], didx)
    pltpu.sync_copy(b1, b1_t)
    pltpu.async_copy(h0.at[rsl], rbuf, hsem)

    def _fill_ones(j, _):
        ones_t[pl.ds(j * L, L)] = jnp.full((L,), 1.0, jnp.float32)
        return 0
    lax.fori_loop(0, CH // L, _fill_ones, 0)

    def _fill_deg(j, _):
        degb[pl.ds(j * L, L)] = jnp.full((L,), 1.0, jnp.float32)
        return 0
    lax.fori_loop(0, RPW // L, _fill_deg, 0)
    pltpu.sync_copy(degb, deg_s.at[rsl])
    plsc.subcore_barrier()

    def _deg_wait():
        pltpu.make_async_copy(ones_t, deg_s.at[didx.at[0]], dsem).wait()

    def _deg_step(k, _):
        pltpu.async_copy(ones_t, deg_s.at[didx.at[k]], dsem, add=True)

        @pl.when(k >= 4)
        def _():
            _deg_wait()
        return 0
    lax.fori_loop(0, K, _deg_step, 0)
    for _ in range(4):
        _deg_wait()
    plsc.subcore_barrier()

    pltpu.sync_copy(deg_s.at[rsl], degb)

    def _rsq_step(j, _):
        sl = pl.ds(j * L, L)
        dinv_t[sl] = _rsqrt16(degb[sl])
        return 0
    lax.fori_loop(0, RPW // L, _rsq_step, 0)

    def _rowloop(fn):
        def _body(j, _):
            dv = dinv_t[pl.ds(j * L, L)]
            for t in range(L):
                fn(j * L + t, dv[t])
            return 0
        lax.fori_loop(0, RPW // L, _body, 0)

    pltpu.make_async_copy(h0.at[rsl], rbuf, hsem).wait()

    def _scale1(r, s):
        rbuf[r, :] = rbuf[r, :] * s
    _rowloop(_scale1)
    pltpu.sync_copy(rbuf, hs1_t.at[rsl])
    pltpu.sync_copy(rbuf, hs1_s.at[rsl])
    pltpu.sync_copy(rbuf, acc1_s.at[rsl])
    plsc.subcore_barrier()

    def _aggregate(table, table_s, acc):
        def _start_g(k, b):
            src = table_s if (b % 2 == 0) else table
            pltpu.async_copy(src.at[sidx.at[k]], rows.at[b], gsem.at[b])

        def _wait_g(b):
            pltpu.make_async_copy(table.at[sidx.at[0]], rows.at[b],
                                  gsem.at[b]).wait()

        def _start_s(k, b):
            pltpu.async_copy(rows.at[b], acc.at[didx.at[k]], ssem.at[b],
                             add=True)

        def _wait_s(b):
            pltpu.make_async_copy(rows.at[b], acc.at[didx.at[0]],
                                  ssem.at[b]).wait()

        half = NSLOT // 2
        for b in range(half):
            _start_g(b, b)

        def _grp(g, _):
            for t in range(NSLOT):
                k = g * NSLOT + t
                _wait_g(t)
                _start_s(k, t)
                bn = (t + half) % NSLOT

                @pl.when(k + half < K)
                def _():
                    @pl.when(k >= half)
                    def _():
                        _wait_s(bn)
                    _start_g(k + half, bn)
            return 0
        lax.fori_loop(0, K // NSLOT, _grp, 0)
        for b in range(NSLOT):
            _wait_s(b)

    _aggregate(hs1_t, hs1_s, acc1_s)
    plsc.subcore_barrier()

    pltpu.sync_copy(acc1_s.at[rsl], rbuf)
    b1v = b1_t[...]

    def _mid(r, s):
        v = jnp.maximum(rbuf[r, :] * s + b1v, 0.0)
        rbuf[r, :] = v * s
    _rowloop(_mid)
    pltpu.sync_copy(rbuf, hs2_t.at[rsl])
    pltpu.sync_copy(rbuf, hs2_s.at[rsl])
    pltpu.sync_copy(rbuf, acc2_s.at[rsl])
    plsc.subcore_barrier()

    _aggregate(hs2_t, hs2_s, acc2_s)
    plsc.subcore_barrier()

    pltpu.sync_copy(acc2_s.at[rsl], rbuf)

    def _scale2(r, s):
        rbuf[r, :] = rbuf[r, :] * s
    _rowloop(_scale2)
    pltpu.sync_copy(rbuf, z_out.at[rsl])


_sc_agg = pl.kernel(
    _sc_body,
    out_type=(
        jax.ShapeDtypeStruct((NPAD, D_HID), jnp.float32),
        jax.ShapeDtypeStruct((NPAD, D_HID), jnp.float32),
        jax.ShapeDtypeStruct((NPAD, D_HID), jnp.float32),
    ),
    mesh=plsc.VectorSubcoreMesh(core_axis_name="c", subcore_axis_name="s",
                                num_cores=1),
    compiler_params=pltpu.CompilerParams(use_tc_tiling_on_sc=False),
    scratch_types=(
        pltpu.VMEM_SHARED((NPAD,), jnp.float32),
        pltpu.VMEM_SHARED((NPAD, D_HID), jnp.float32),
        pltpu.VMEM_SHARED((NPAD, D_HID), jnp.float32),
        pltpu.VMEM_SHARED((NPAD, D_HID), jnp.float32),
        pltpu.VMEM_SHARED((NPAD, D_HID), jnp.float32),
        pltpu.VMEM((K, CH), jnp.int32),
        pltpu.VMEM((K, CH), jnp.int32),
        pltpu.VMEM((NSLOT, CH, D_HID), jnp.float32),
        pltpu.VMEM((RPW, D_HID), jnp.float32),
        pltpu.VMEM((RPW,), jnp.float32),
        pltpu.VMEM((RPW,), jnp.float32),
        pltpu.VMEM((CH,), jnp.float32),
        pltpu.VMEM((D_HID,), jnp.float32),
        pltpu.SemaphoreType.DMA((NSLOT,)),
        pltpu.SemaphoreType.DMA((NSLOT,)),
        pltpu.SemaphoreType.DMA,
        pltpu.SemaphoreType.DMA,
    ),
)


def kernel(x, edge_index, W1, b1, W2, b2):
    ei = edge_index.astype(jnp.int32)
    pad = jnp.full((EPAD - E,), N, jnp.int32)
    srcp = jnp.concatenate([ei[0], pad]).reshape(NS, K, CH)
    dstp = jnp.concatenate([ei[1], pad]).reshape(NS, K, CH)

    h0 = pl.pallas_call(
        _mm1_body,
        grid=(5,),
        in_specs=[
            pl.BlockSpec((2000, D_IN), lambda i: (i, 0)),
            pl.BlockSpec((D_IN, D_HID), lambda i: (0, 0)),
        ],
        out_specs=pl.BlockSpec((2000, D_HID), lambda i: (i, 0)),
        out_shape=jax.ShapeDtypeStruct((NPAD, D_HID), jnp.float32),
    )(x, W1)

    z, _, _ = _sc_agg(h0, srcp, dstp, b1)

    out = pl.pallas_call(
        _head_body,
        grid=(5,),
        in_specs=[
            pl.BlockSpec((2000, D_HID), lambda i: (i, 0)),
            pl.BlockSpec((D_HID, N_CLASSES), lambda i: (0, 0)),
            pl.BlockSpec((1, N_CLASSES), lambda i: (0, 0)),
        ],
        out_specs=pl.BlockSpec((2000, N_CLASSES), lambda i: (i, 0)),
        out_shape=jax.ShapeDtypeStruct((N, N_CLASSES), jnp.float32),
    )(z, W2, b2.reshape(1, N_CLASSES))
    return out


# speedup vs baseline: 60.9181x; 1.1336x over previous
import functools

import jax
import jax.numpy as jnp
from jax import lax
from jax.experimental import pallas as pl
from jax.experimental.pallas import tpu as pltpu
from jax.experimental.pallas import tpu_sc as plsc

N = 10000
D_IN = 128
D_HID = 16
N_CLASSES = 40
E = 320000

NS = 16
L = 16
NPAD = 10240
RPW = NPAD // NS
CH = 256
EPW = 20480
K = EPW // CH
NSLOT = 8
EPAD = EPW * NS


def _mm1_body(x_ref, w_ref, o_ref):
    o_ref[...] = jnp.dot(x_ref[...], w_ref[...],
                         preferred_element_type=jnp.float32)


def _head_body(z_ref, w_ref, b_ref, o_ref):
    o = jnp.dot(z_ref[...], w_ref[...],
                preferred_element_type=jnp.float32) + b_ref[...]
    m = jnp.max(o, axis=1, keepdims=True)
    s = jnp.sum(jnp.exp(o - m), axis=1, keepdims=True)
    o_ref[...] = o - m - jnp.log(s)


def _rsqrt16(d):
    i = lax.bitcast_convert_type(d, jnp.int32)
    i = jnp.int32(0x5F3759DF) - lax.shift_right_logical(i, 1)
    y = lax.bitcast_convert_type(i, jnp.float32)
    for _ in range(4):
        y = y * (1.5 - 0.5 * d * y * y)
    return y


def _sc_body(h0, srcp, dstp, b1, z_out, hs1_t, hs2_t,
             deg_s, acc1_s, acc2_s, hs1_s, hs2_s,
             sidx, didx, rows, rbuf, dinv_t, degb, ones_t, b1_t,
             gsem, ssem, dsem, hsem):
    wid = lax.axis_index("s")
    rbase = wid * RPW
    rsl = pl.ds(rbase, RPW)

    pltpu.sync_copy(srcp.at[wid], sidx)
    pltpu.sync_copy(dstp.at[wid], didx)
    pltpu.sync_copy(b1, b1_t)
    pltpu.async_copy(h0.at[rsl], rbuf, hsem)

    def _fill_ones(j, _):
        ones_t[pl.ds(j * L, L)] = jnp.full((L,), 1.0, jnp.float32)
        return 0
    lax.fori_loop(0, CH // L, _fill_ones, 0)

    def _fill_deg(j, _):
        degb[pl.ds(j * L, L)] = jnp.full((L,), 1.0, jnp.float32)
        return 0
    lax.fori_loop(0, RPW // L, _fill_deg, 0)
    pltpu.sync_copy(degb, deg_s.at[rsl])
    plsc.subcore_barrier()

    def _deg_wait():
        pltpu.make_async_copy(ones_t, deg_s.at[didx.at[0]], dsem).wait()

    def _deg_step(k, _):
        pltpu.async_copy(ones_t, deg_s.at[didx.at[k]], dsem, add=True)

        @pl.when(k >= 4)
        def _():
            _deg_wait()
        return 0
    lax.fori_loop(0, K, _deg_step, 0)
    for _ in range(4):
        _deg_wait()
    plsc.subcore_barrier()

    pltpu.sync_copy(deg_s.at[rsl], degb)

    def _rsq_step(j, _):
        sl = pl.ds(j * L, L)
        dinv_t[sl] = _rsqrt16(degb[sl])
        return 0
    lax.fori_loop(0, RPW // L, _rsq_step, 0)

    def _rowloop(fn):
        def _body(j, _):
            dv = dinv_t[pl.ds(j * L, L)]
            for t in range(L):
                fn(j * L + t, dv[t])
            return 0
        lax.fori_loop(0, RPW // L, _body, 0)

    pltpu.make_async_copy(h0.at[rsl], rbuf, hsem).wait()

    def _scale1(r, s):
        rbuf[r, :] = rbuf[r, :] * s
    _rowloop(_scale1)
    pltpu.sync_copy(rbuf, hs1_t.at[rsl])
    pltpu.sync_copy(rbuf, hs1_s.at[rsl])
    pltpu.sync_copy(rbuf, acc1_s.at[rsl])
    plsc.subcore_barrier()

    def _aggregate(table, table_s, acc):
        def _start_g(k, b):
            src = table_s if (b % 4 < 3) else table
            pltpu.async_copy(src.at[sidx.at[k]], rows.at[b], gsem.at[b])

        def _wait_g(b):
            pltpu.make_async_copy(table.at[sidx.at[0]], rows.at[b],
                                  gsem.at[b]).wait()

        def _start_s(k, b):
            pltpu.async_copy(rows.at[b], acc.at[didx.at[k]], ssem.at[b],
                             add=True)

        def _wait_s(b):
            pltpu.make_async_copy(rows.at[b], acc.at[didx.at[0]],
                                  ssem.at[b]).wait()

        half = NSLOT // 2
        for b in range(half):
            _start_g(b, b)

        def _grp(g, _):
            for t in range(NSLOT):
                k = g * NSLOT + t
                _wait_g(t)
                _start_s(k, t)
                bn = (t + half) % NSLOT

                @pl.when(k + half < K)
                def _():
                    @pl.when(k >= half)
                    def _():
                        _wait_s(bn)
                    _start_g(k + half, bn)
            return 0
        lax.fori_loop(0, K // NSLOT, _grp, 0)
        for b in range(NSLOT):
            _wait_s(b)

    _aggregate(hs1_t, hs1_s, acc1_s)
    plsc.subcore_barrier()

    pltpu.sync_copy(acc1_s.at[rsl], rbuf)
    b1v = b1_t[...]

    def _mid(r, s):
        v = jnp.maximum(rbuf[r, :] * s + b1v, 0.0)
        rbuf[r, :] = v * s
    _rowloop(_mid)
    pltpu.sync_copy(rbuf, hs2_t.at[rsl])
    pltpu.sync_copy(rbuf, hs2_s.at[rsl])
    pltpu.sync_copy(rbuf, acc2_s.at[rsl])
    plsc.subcore_barrier()

    _aggregate(hs2_t, hs2_s, acc2_s)
    plsc.subcore_barrier()

    pltpu.sync_copy(acc2_s.at[rsl], rbuf)

    def _scale2(r, s):
        rbuf[r, :] = rbuf[r, :] * s
    _rowloop(_scale2)
    pltpu.sync_copy(rbuf, z_out.at[rsl])


_sc_agg = pl.kernel(
    _sc_body,
    out_type=(
        jax.ShapeDtypeStruct((NPAD, D_HID), jnp.float32),
        jax.ShapeDtypeStruct((NPAD, D_HID), jnp.float32),
        jax.ShapeDtypeStruct((NPAD, D_HID), jnp.float32),
    ),
    mesh=plsc.VectorSubcoreMesh(core_axis_name="c", subcore_axis_name="s",
                                num_cores=1),
    compiler_params=pltpu.CompilerParams(use_tc_tiling_on_sc=False),
    scratch_types=(
        pltpu.VMEM_SHARED((NPAD,), jnp.float32),
        pltpu.VMEM_SHARED((NPAD, D_HID), jnp.float32),
        pltpu.VMEM_SHARED((NPAD, D_HID), jnp.float32),
        pltpu.VMEM_SHARED((NPAD, D_HID), jnp.float32),
        pltpu.VMEM_SHARED((NPAD, D_HID), jnp.float32),
        pltpu.VMEM((K, CH), jnp.int32),
        pltpu.VMEM((K, CH), jnp.int32),
        pltpu.VMEM((NSLOT, CH, D_HID), jnp.float32),
        pltpu.VMEM((RPW, D_HID), jnp.float32),
        pltpu.VMEM((RPW,), jnp.float32),
        pltpu.VMEM((RPW,), jnp.float32),
        pltpu.VMEM((CH,), jnp.float32),
        pltpu.VMEM((D_HID,), jnp.float32),
        pltpu.SemaphoreType.DMA((NSLOT,)),
        pltpu.SemaphoreType.DMA((NSLOT,)),
        pltpu.SemaphoreType.DMA,
        pltpu.SemaphoreType.DMA,
    ),
)


def kernel(x, edge_index, W1, b1, W2, b2):
    ei = edge_index.astype(jnp.int32)
    pad = jnp.full((EPAD - E,), N, jnp.int32)
    srcp = jnp.concatenate([ei[0], pad]).reshape(NS, K, CH)
    dstp = jnp.concatenate([ei[1], pad]).reshape(NS, K, CH)

    h0 = pl.pallas_call(
        _mm1_body,
        grid=(5,),
        in_specs=[
            pl.BlockSpec((2000, D_IN), lambda i: (i, 0)),
            pl.BlockSpec((D_IN, D_HID), lambda i: (0, 0)),
        ],
        out_specs=pl.BlockSpec((2000, D_HID), lambda i: (i, 0)),
        out_shape=jax.ShapeDtypeStruct((NPAD, D_HID), jnp.float32),
    )(x, W1)

    z, _, _ = _sc_agg(h0, srcp, dstp, b1)

    out = pl.pallas_call(
        _head_body,
        grid=(5,),
        in_specs=[
            pl.BlockSpec((2000, D_HID), lambda i: (i, 0)),
            pl.BlockSpec((D_HID, N_CLASSES), lambda i: (0, 0)),
            pl.BlockSpec((1, N_CLASSES), lambda i: (0, 0)),
        ],
        out_specs=pl.BlockSpec((2000, N_CLASSES), lambda i: (i, 0)),
        out_shape=jax.ShapeDtypeStruct((N, N_CLASSES), jnp.float32),
    )(z, W2, b2.reshape(1, N_CLASSES))
    return out


# speedup vs baseline: 62.8154x; 1.0311x over previous
import functools

import jax
import jax.numpy as jnp
from jax import lax
from jax.experimental import pallas as pl
from jax.experimental.pallas import tpu as pltpu
from jax.experimental.pallas import tpu_sc as plsc

N = 10000
D_IN = 128
D_HID = 16
N_CLASSES = 40
E = 320000

NS = 16
L = 16
NPAD = 10240
RPW = NPAD // NS
CH = 256
EPW = 20480
K = EPW // CH
NSLOT = 8
EPAD = EPW * NS


def _mm1_body(x_ref, w_ref, o_ref):
    o_ref[...] = jnp.dot(x_ref[...], w_ref[...],
                         preferred_element_type=jnp.float32)


def _head_body(z_ref, w_ref, b_ref, o_ref):
    o = jnp.dot(z_ref[...], w_ref[...],
                preferred_element_type=jnp.float32) + b_ref[...]
    m = jnp.max(o, axis=1, keepdims=True)
    s = jnp.sum(jnp.exp(o - m), axis=1, keepdims=True)
    o_ref[...] = o - m - jnp.log(s)


def _rsqrt16(d):
    i = lax.bitcast_convert_type(d, jnp.int32)
    i = jnp.int32(0x5F3759DF) - lax.shift_right_logical(i, 1)
    y = lax.bitcast_convert_type(i, jnp.float32)
    for _ in range(4):
        y = y * (1.5 - 0.5 * d * y * y)
    return y


def _sc_body(h0, srcp, dstp, b1, z_out, hs1_t, hs2_t,
             deg_s, acc1_s, acc2_s, hs1_s, hs2_s,
             sidx, didx, rows, rbuf, dinv_t, degb, ones_t, b1_t,
             gsem, ssem, dsem, hsem):
    wid = lax.axis_index("s")
    rbase = wid * RPW
    rsl = pl.ds(rbase, RPW)

    pltpu.sync_copy(srcp.at[wid], sidx)
    pltpu.sync_copy(dstp.at[wid], didx)
    pltpu.sync_copy(b1, b1_t)
    pltpu.async_copy(h0.at[rsl], rbuf, hsem)

    def _fill_ones(j, _):
        ones_t[pl.ds(j * L, L)] = jnp.full((L,), 1.0, jnp.float32)
        return 0
    lax.fori_loop(0, CH // L, _fill_ones, 0)

    def _fill_deg(j, _):
        degb[pl.ds(j * L, L)] = jnp.full((L,), 1.0, jnp.float32)
        return 0
    lax.fori_loop(0, RPW // L, _fill_deg, 0)
    pltpu.sync_copy(degb, deg_s.at[rsl])
    plsc.subcore_barrier()

    def _deg_wait():
        pltpu.make_async_copy(ones_t, deg_s.at[didx.at[0]], dsem).wait()

    def _deg_step(k, _):
        pltpu.async_copy(ones_t, deg_s.at[didx.at[k]], dsem, add=True)

        @pl.when(k >= 4)
        def _():
            _deg_wait()
        return 0
    lax.fori_loop(0, K, _deg_step, 0)
    for _ in range(4):
        _deg_wait()
    plsc.subcore_barrier()

    pltpu.sync_copy(deg_s.at[rsl], degb)

    def _rsq_step(j, _):
        sl = pl.ds(j * L, L)
        dinv_t[sl] = _rsqrt16(degb[sl])
        return 0
    lax.fori_loop(0, RPW // L, _rsq_step, 0)

    def _rowloop(fn):
        def _body(j, _):
            dv = dinv_t[pl.ds(j * L, L)]
            for t in range(L):
                fn(j * L + t, dv[t])
            return 0
        lax.fori_loop(0, RPW // L, _body, 0)

    pltpu.make_async_copy(h0.at[rsl], rbuf, hsem).wait()

    def _scale1(r, s):
        rbuf[r, :] = rbuf[r, :] * s
    _rowloop(_scale1)
    pltpu.sync_copy(rbuf, hs1_t.at[rsl])
    pltpu.sync_copy(rbuf, hs1_s.at[rsl])
    pltpu.sync_copy(rbuf, acc1_s.at[rsl])
    plsc.subcore_barrier()

    def _aggregate(table, table_s, acc):
        def _start_g(k, b):
            src = table_s if (b % 8 < 7) else table
            pltpu.async_copy(src.at[sidx.at[k]], rows.at[b], gsem.at[b])

        def _wait_g(b):
            pltpu.make_async_copy(table.at[sidx.at[0]], rows.at[b],
                                  gsem.at[b]).wait()

        def _start_s(k, b):
            pltpu.async_copy(rows.at[b], acc.at[didx.at[k]], ssem.at[b],
                             add=True)

        def _wait_s(b):
            pltpu.make_async_copy(rows.at[b], acc.at[didx.at[0]],
                                  ssem.at[b]).wait()

        half = NSLOT // 2
        for b in range(half):
            _start_g(b, b)

        def _grp(g, _):
            for t in range(NSLOT):
                k = g * NSLOT + t
                _wait_g(t)
                _start_s(k, t)
                bn = (t + half) % NSLOT

                @pl.when(k + half < K)
                def _():
                    @pl.when(k >= half)
                    def _():
                        _wait_s(bn)
                    _start_g(k + half, bn)
            return 0
        lax.fori_loop(0, K // NSLOT, _grp, 0)
        for b in range(NSLOT):
            _wait_s(b)

    _aggregate(hs1_t, hs1_s, acc1_s)
    plsc.subcore_barrier()

    pltpu.sync_copy(acc1_s.at[rsl], rbuf)
    b1v = b1_t[...]

    def _mid(r, s):
        v = jnp.maximum(rbuf[r, :] * s + b1v, 0.0)
        rbuf[r, :] = v * s
    _rowloop(_mid)
    pltpu.sync_copy(rbuf, hs2_t.at[rsl])
    pltpu.sync_copy(rbuf, hs2_s.at[rsl])
    pltpu.sync_copy(rbuf, acc2_s.at[rsl])
    plsc.subcore_barrier()

    _aggregate(hs2_t, hs2_s, acc2_s)
    plsc.subcore_barrier()

    pltpu.sync_copy(acc2_s.at[rsl], rbuf)

    def _scale2(r, s):
        rbuf[r, :] = rbuf[r, :] * s
    _rowloop(_scale2)
    pltpu.sync_copy(rbuf, z_out.at[rsl])


_sc_agg = pl.kernel(
    _sc_body,
    out_type=(
        jax.ShapeDtypeStruct((NPAD, D_HID), jnp.float32),
        jax.ShapeDtypeStruct((NPAD, D_HID), jnp.float32),
        jax.ShapeDtypeStruct((NPAD, D_HID), jnp.float32),
    ),
    mesh=plsc.VectorSubcoreMesh(core_axis_name="c", subcore_axis_name="s",
                                num_cores=1),
    compiler_params=pltpu.CompilerParams(use_tc_tiling_on_sc=False),
    scratch_types=(
        pltpu.VMEM_SHARED((NPAD,), jnp.float32),
        pltpu.VMEM_SHARED((NPAD, D_HID), jnp.float32),
        pltpu.VMEM_SHARED((NPAD, D_HID), jnp.float32),
        pltpu.VMEM_SHARED((NPAD, D_HID), jnp.float32),
        pltpu.VMEM_SHARED((NPAD, D_HID), jnp.float32),
        pltpu.VMEM((K, CH), jnp.int32),
        pltpu.VMEM((K, CH), jnp.int32),
        pltpu.VMEM((NSLOT, CH, D_HID), jnp.float32),
        pltpu.VMEM((RPW, D_HID), jnp.float32),
        pltpu.VMEM((RPW,), jnp.float32),
        pltpu.VMEM((RPW,), jnp.float32),
        pltpu.VMEM((CH,), jnp.float32),
        pltpu.VMEM((D_HID,), jnp.float32),
        pltpu.SemaphoreType.DMA((NSLOT,)),
        pltpu.SemaphoreType.DMA((NSLOT,)),
        pltpu.SemaphoreType.DMA,
        pltpu.SemaphoreType.DMA,
    ),
)


def kernel(x, edge_index, W1, b1, W2, b2):
    ei = edge_index.astype(jnp.int32)
    pad = jnp.full((EPAD - E,), N, jnp.int32)
    srcp = jnp.concatenate([ei[0], pad]).reshape(NS, K, CH)
    dstp = jnp.concatenate([ei[1], pad]).reshape(NS, K, CH)

    h0 = pl.pallas_call(
        _mm1_body,
        grid=(5,),
        in_specs=[
            pl.BlockSpec((2000, D_IN), lambda i: (i, 0)),
            pl.BlockSpec((D_IN, D_HID), lambda i: (0, 0)),
        ],
        out_specs=pl.BlockSpec((2000, D_HID), lambda i: (i, 0)),
        out_shape=jax.ShapeDtypeStruct((NPAD, D_HID), jnp.float32),
    )(x, W1)

    z, _, _ = _sc_agg(h0, srcp, dstp, b1)

    out = pl.pallas_call(
        _head_body,
        grid=(5,),
        in_specs=[
            pl.BlockSpec((2000, D_HID), lambda i: (i, 0)),
            pl.BlockSpec((D_HID, N_CLASSES), lambda i: (0, 0)),
            pl.BlockSpec((1, N_CLASSES), lambda i: (0, 0)),
        ],
        out_specs=pl.BlockSpec((2000, N_CLASSES), lambda i: (i, 0)),
        out_shape=jax.ShapeDtypeStruct((N, N_CLASSES), jnp.float32),
    )(z, W2, b2.reshape(1, N_CLASSES))
    return out


# speedup vs baseline: 64.7133x; 1.0302x over previous
import functools

import jax
import jax.numpy as jnp
from jax import lax
from jax.experimental import pallas as pl
from jax.experimental.pallas import tpu as pltpu
from jax.experimental.pallas import tpu_sc as plsc

N = 10000
D_IN = 128
D_HID = 16
N_CLASSES = 40
E = 320000

NS = 16
L = 16
NPAD = 10240
RPW = NPAD // NS
CH = 256
EPW = 20480
K = EPW // CH
NSLOT = 8
EPAD = EPW * NS


def _mm1_body(x_ref, w_ref, o_ref):
    o_ref[...] = jnp.dot(x_ref[...], w_ref[...],
                         preferred_element_type=jnp.float32)


def _head_body(z_ref, w_ref, b_ref, o_ref):
    o = jnp.dot(z_ref[...], w_ref[...],
                preferred_element_type=jnp.float32) + b_ref[...]
    m = jnp.max(o, axis=1, keepdims=True)
    s = jnp.sum(jnp.exp(o - m), axis=1, keepdims=True)
    o_ref[...] = o - m - jnp.log(s)


def _rsqrt16(d):
    i = lax.bitcast_convert_type(d, jnp.int32)
    i = jnp.int32(0x5F3759DF) - lax.shift_right_logical(i, 1)
    y = lax.bitcast_convert_type(i, jnp.float32)
    for _ in range(4):
        y = y * (1.5 - 0.5 * d * y * y)
    return y


def _sc_body(h0, srcp, dstp, b1, z_out, hs1_t, hs2_t,
             deg_s, acc1_s, acc2_s, hs1_s, hs2_s,
             sidx, didx, rows, rbuf, dinv_t, degb, ones_t, b1_t,
             gsem, ssem, dsem, hsem):
    wid = lax.axis_index("s")
    rbase = wid * RPW
    rsl = pl.ds(rbase, RPW)

    pltpu.sync_copy(srcp.at[wid], sidx)
    pltpu.sync_copy(dstp.at[wid], didx)
    pltpu.sync_copy(b1, b1_t)
    pltpu.async_copy(h0.at[rsl], rbuf, hsem)

    def _fill_ones(j, _):
        ones_t[pl.ds(j * L, L)] = jnp.full((L,), 1.0, jnp.float32)
        return 0
    lax.fori_loop(0, CH // L, _fill_ones, 0)

    def _fill_deg(j, _):
        degb[pl.ds(j * L, L)] = jnp.full((L,), 1.0, jnp.float32)
        return 0
    lax.fori_loop(0, RPW // L, _fill_deg, 0)
    pltpu.sync_copy(degb, deg_s.at[rsl])
    plsc.subcore_barrier()

    def _deg_wait():
        pltpu.make_async_copy(ones_t, deg_s.at[didx.at[0]], dsem).wait()

    def _deg_step(k, _):
        pltpu.async_copy(ones_t, deg_s.at[didx.at[k]], dsem, add=True)

        @pl.when(k >= 4)
        def _():
            _deg_wait()
        return 0
    lax.fori_loop(0, K, _deg_step, 0)
    for _ in range(4):
        _deg_wait()
    plsc.subcore_barrier()

    pltpu.sync_copy(deg_s.at[rsl], degb)

    def _rsq_step(j, _):
        sl = pl.ds(j * L, L)
        dinv_t[sl] = _rsqrt16(degb[sl])
        return 0
    lax.fori_loop(0, RPW // L, _rsq_step, 0)

    def _rowloop(fn):
        def _body(j, _):
            dv = dinv_t[pl.ds(j * L, L)]
            for t in range(L):
                fn(j * L + t, dv[t])
            return 0
        lax.fori_loop(0, RPW // L, _body, 0)

    pltpu.make_async_copy(h0.at[rsl], rbuf, hsem).wait()

    def _scale1(r, s):
        rbuf[r, :] = rbuf[r, :] * s
    _rowloop(_scale1)
    pltpu.sync_copy(rbuf, hs1_t.at[rsl])
    pltpu.sync_copy(rbuf, hs1_s.at[rsl])
    pltpu.sync_copy(rbuf, acc1_s.at[rsl])
    plsc.subcore_barrier()

    def _aggregate(table, table_s, acc):
        def _start_g(k, b):
            src = table_s
            pltpu.async_copy(src.at[sidx.at[k]], rows.at[b], gsem.at[b])

        def _wait_g(b):
            pltpu.make_async_copy(table.at[sidx.at[0]], rows.at[b],
                                  gsem.at[b]).wait()

        def _start_s(k, b):
            pltpu.async_copy(rows.at[b], acc.at[didx.at[k]], ssem.at[b],
                             add=True)

        def _wait_s(b):
            pltpu.make_async_copy(rows.at[b], acc.at[didx.at[0]],
                                  ssem.at[b]).wait()

        half = NSLOT // 2
        for b in range(half):
            _start_g(b, b)

        def _grp(g, _):
            for t in range(NSLOT):
                k = g * NSLOT + t
                _wait_g(t)
                _start_s(k, t)
                bn = (t + half) % NSLOT

                @pl.when(k + half < K)
                def _():
                    @pl.when(k >= half)
                    def _():
                        _wait_s(bn)
                    _start_g(k + half, bn)
            return 0
        lax.fori_loop(0, K // NSLOT, _grp, 0)
        for b in range(NSLOT):
            _wait_s(b)

    _aggregate(hs1_t, hs1_s, acc1_s)
    plsc.subcore_barrier()

    pltpu.sync_copy(acc1_s.at[rsl], rbuf)
    b1v = b1_t[...]

    def _mid(r, s):
        v = jnp.maximum(rbuf[r, :] * s + b1v, 0.0)
        rbuf[r, :] = v * s
    _rowloop(_mid)
    pltpu.sync_copy(rbuf, hs2_t.at[rsl])
    pltpu.sync_copy(rbuf, hs2_s.at[rsl])
    pltpu.sync_copy(rbuf, acc2_s.at[rsl])
    plsc.subcore_barrier()

    _aggregate(hs2_t, hs2_s, acc2_s)
    plsc.subcore_barrier()

    pltpu.sync_copy(acc2_s.at[rsl], rbuf)

    def _scale2(r, s):
        rbuf[r, :] = rbuf[r, :] * s
    _rowloop(_scale2)
    pltpu.sync_copy(rbuf, z_out.at[rsl])


_sc_agg = pl.kernel(
    _sc_body,
    out_type=(
        jax.ShapeDtypeStruct((NPAD, D_HID), jnp.float32),
        jax.ShapeDtypeStruct((NPAD, D_HID), jnp.float32),
        jax.ShapeDtypeStruct((NPAD, D_HID), jnp.float32),
    ),
    mesh=plsc.VectorSubcoreMesh(core_axis_name="c", subcore_axis_name="s",
                                num_cores=1),
    compiler_params=pltpu.CompilerParams(use_tc_tiling_on_sc=False),
    scratch_types=(
        pltpu.VMEM_SHARED((NPAD,), jnp.float32),
        pltpu.VMEM_SHARED((NPAD, D_HID), jnp.float32),
        pltpu.VMEM_SHARED((NPAD, D_HID), jnp.float32),
        pltpu.VMEM_SHARED((NPAD, D_HID), jnp.float32),
        pltpu.VMEM_SHARED((NPAD, D_HID), jnp.float32),
        pltpu.VMEM((K, CH), jnp.int32),
        pltpu.VMEM((K, CH), jnp.int32),
        pltpu.VMEM((NSLOT, CH, D_HID), jnp.float32),
        pltpu.VMEM((RPW, D_HID), jnp.float32),
        pltpu.VMEM((RPW,), jnp.float32),
        pltpu.VMEM((RPW,), jnp.float32),
        pltpu.VMEM((CH,), jnp.float32),
        pltpu.VMEM((D_HID,), jnp.float32),
        pltpu.SemaphoreType.DMA((NSLOT,)),
        pltpu.SemaphoreType.DMA((NSLOT,)),
        pltpu.SemaphoreType.DMA,
        pltpu.SemaphoreType.DMA,
    ),
)


def kernel(x, edge_index, W1, b1, W2, b2):
    ei = edge_index.astype(jnp.int32)
    pad = jnp.full((EPAD - E,), N, jnp.int32)
    srcp = jnp.concatenate([ei[0], pad]).reshape(NS, K, CH)
    dstp = jnp.concatenate([ei[1], pad]).reshape(NS, K, CH)

    h0 = pl.pallas_call(
        _mm1_body,
        grid=(5,),
        in_specs=[
            pl.BlockSpec((2000, D_IN), lambda i: (i, 0)),
            pl.BlockSpec((D_IN, D_HID), lambda i: (0, 0)),
        ],
        out_specs=pl.BlockSpec((2000, D_HID), lambda i: (i, 0)),
        out_shape=jax.ShapeDtypeStruct((NPAD, D_HID), jnp.float32),
    )(x, W1)

    z, _, _ = _sc_agg(h0, srcp, dstp, b1)

    out = pl.pallas_call(
        _head_body,
        grid=(5,),
        in_specs=[
            pl.BlockSpec((2000, D_HID), lambda i: (i, 0)),
            pl.BlockSpec((D_HID, N_CLASSES), lambda i: (0, 0)),
            pl.BlockSpec((1, N_CLASSES), lambda i: (0, 0)),
        ],
        out_specs=pl.BlockSpec((2000, N_CLASSES), lambda i: (i, 0)),
        out_shape=jax.ShapeDtypeStruct((N, N_CLASSES), jnp.float32),
    )(z, W2, b2.reshape(1, N_CLASSES))
    return out


# speedup vs baseline: 65.1917x; 1.0074x over previous
import functools

import jax
import jax.numpy as jnp
from jax import lax
from jax.experimental import pallas as pl
from jax.experimental.pallas import tpu as pltpu
from jax.experimental.pallas import tpu_sc as plsc

N = 10000
D_IN = 128
D_HID = 16
N_CLASSES = 40
E = 320000

NS = 16
L = 16
NPAD = 10240
RPW = NPAD // NS
CH = 256
EPW = 20480
K = EPW // CH
NSLOT = 8
EPAD = EPW * NS


def _mm1_body(x_ref, w_ref, o_ref):
    o_ref[...] = jnp.dot(x_ref[...], w_ref[...],
                         preferred_element_type=jnp.float32)


def _head_body(z_ref, w_ref, b_ref, o_ref):
    o = jnp.dot(z_ref[...], w_ref[...],
                preferred_element_type=jnp.float32) + b_ref[...]
    m = jnp.max(o, axis=1, keepdims=True)
    s = jnp.sum(jnp.exp(o - m), axis=1, keepdims=True)
    o_ref[...] = o - m - jnp.log(s)


def _rsqrt16(d):
    i = lax.bitcast_convert_type(d, jnp.int32)
    i = jnp.int32(0x5F3759DF) - lax.shift_right_logical(i, 1)
    y = lax.bitcast_convert_type(i, jnp.float32)
    for _ in range(4):
        y = y * (1.5 - 0.5 * d * y * y)
    return y


def _sc_body(h0, srcp, dstp, b1, z_out,
             deg_s, acc1_s, acc2_s, hs1_s, hs2_s,
             sidx, didx, rows, rbuf, dinv_t, degb, ones_t, b1_t,
             gsem, ssem, dsem, hsem):
    wid = lax.axis_index("s")
    rbase = wid * RPW
    rsl = pl.ds(rbase, RPW)

    pltpu.sync_copy(srcp.at[wid], sidx)
    pltpu.sync_copy(dstp.at[wid], didx)
    pltpu.sync_copy(b1, b1_t)
    pltpu.async_copy(h0.at[rsl], rbuf, hsem)

    def _fill_ones(j, _):
        ones_t[pl.ds(j * L, L)] = jnp.full((L,), 1.0, jnp.float32)
        return 0
    lax.fori_loop(0, CH // L, _fill_ones, 0)

    def _fill_deg(j, _):
        degb[pl.ds(j * L, L)] = jnp.full((L,), 1.0, jnp.float32)
        return 0
    lax.fori_loop(0, RPW // L, _fill_deg, 0)
    pltpu.sync_copy(degb, deg_s.at[rsl])
    plsc.subcore_barrier()

    def _deg_wait():
        pltpu.make_async_copy(ones_t, deg_s.at[didx.at[0]], dsem).wait()

    def _deg_step(k, _):
        pltpu.async_copy(ones_t, deg_s.at[didx.at[k]], dsem, add=True)

        @pl.when(k >= 4)
        def _():
            _deg_wait()
        return 0
    lax.fori_loop(0, K, _deg_step, 0)
    for _ in range(4):
        _deg_wait()
    plsc.subcore_barrier()

    pltpu.sync_copy(deg_s.at[rsl], degb)

    def _rsq_step(j, _):
        sl = pl.ds(j * L, L)
        dinv_t[sl] = _rsqrt16(degb[sl])
        return 0
    lax.fori_loop(0, RPW // L, _rsq_step, 0)

    def _rowloop(fn):
        def _body(j, _):
            dv = dinv_t[pl.ds(j * L, L)]
            for t in range(L):
                fn(j * L + t, dv[t])
            return 0
        lax.fori_loop(0, RPW // L, _body, 0)

    pltpu.make_async_copy(h0.at[rsl], rbuf, hsem).wait()

    def _scale1(r, s):
        rbuf[r, :] = rbuf[r, :] * s
    _rowloop(_scale1)
    pltpu.sync_copy(rbuf, hs1_s.at[rsl])
    pltpu.sync_copy(rbuf, acc1_s.at[rsl])
    plsc.subcore_barrier()

    def _aggregate(table_s, acc):
        def _start_g(k, b):
            pltpu.async_copy(table_s.at[sidx.at[k]], rows.at[b], gsem.at[b])

        def _wait_g(b):
            pltpu.make_async_copy(table_s.at[sidx.at[0]], rows.at[b],
                                  gsem.at[b]).wait()

        def _start_s(k, b):
            pltpu.async_copy(rows.at[b], acc.at[didx.at[k]], ssem.at[b],
                             add=True)

        def _wait_s(b):
            pltpu.make_async_copy(rows.at[b], acc.at[didx.at[0]],
                                  ssem.at[b]).wait()

        half = NSLOT // 2
        for b in range(half):
            _start_g(b, b)

        def _grp(g, _):
            for t in range(NSLOT):
                k = g * NSLOT + t
                _wait_g(t)
                _start_s(k, t)
                bn = (t + half) % NSLOT

                @pl.when(k + half < K)
                def _():
                    @pl.when(k >= half)
                    def _():
                        _wait_s(bn)
                    _start_g(k + half, bn)
            return 0
        lax.fori_loop(0, K // NSLOT, _grp, 0)
        for b in range(NSLOT):
            _wait_s(b)

    _aggregate(hs1_s, acc1_s)
    plsc.subcore_barrier()

    pltpu.sync_copy(acc1_s.at[rsl], rbuf)
    b1v = b1_t[...]

    def _mid(r, s):
        v = jnp.maximum(rbuf[r, :] * s + b1v, 0.0)
        rbuf[r, :] = v * s
    _rowloop(_mid)
    pltpu.sync_copy(rbuf, hs2_s.at[rsl])
    pltpu.sync_copy(rbuf, acc2_s.at[rsl])
    plsc.subcore_barrier()

    _aggregate(hs2_s, acc2_s)
    plsc.subcore_barrier()

    pltpu.sync_copy(acc2_s.at[rsl], rbuf)

    def _scale2(r, s):
        rbuf[r, :] = rbuf[r, :] * s
    _rowloop(_scale2)
    pltpu.sync_copy(rbuf, z_out.at[rsl])


_sc_agg = pl.kernel(
    _sc_body,
    out_type=jax.ShapeDtypeStruct((NPAD, D_HID), jnp.float32),
    mesh=plsc.VectorSubcoreMesh(core_axis_name="c", subcore_axis_name="s",
                                num_cores=1),
    compiler_params=pltpu.CompilerParams(use_tc_tiling_on_sc=False),
    scratch_types=(
        pltpu.VMEM_SHARED((NPAD,), jnp.float32),
        pltpu.VMEM_SHARED((NPAD, D_HID), jnp.float32),
        pltpu.VMEM_SHARED((NPAD, D_HID), jnp.float32),
        pltpu.VMEM_SHARED((NPAD, D_HID), jnp.float32),
        pltpu.VMEM_SHARED((NPAD, D_HID), jnp.float32),
        pltpu.VMEM((K, CH), jnp.int32),
        pltpu.VMEM((K, CH), jnp.int32),
        pltpu.VMEM((NSLOT, CH, D_HID), jnp.float32),
        pltpu.VMEM((RPW, D_HID), jnp.float32),
        pltpu.VMEM((RPW,), jnp.float32),
        pltpu.VMEM((RPW,), jnp.float32),
        pltpu.VMEM((CH,), jnp.float32),
        pltpu.VMEM((D_HID,), jnp.float32),
        pltpu.SemaphoreType.DMA((NSLOT,)),
        pltpu.SemaphoreType.DMA((NSLOT,)),
        pltpu.SemaphoreType.DMA,
        pltpu.SemaphoreType.DMA,
    ),
)


def kernel(x, edge_index, W1, b1, W2, b2):
    ei = edge_index.astype(jnp.int32)
    pad = jnp.full((EPAD - E,), N, jnp.int32)
    srcp = jnp.concatenate([ei[0], pad]).reshape(NS, K, CH)
    dstp = jnp.concatenate([ei[1], pad]).reshape(NS, K, CH)

    h0 = pl.pallas_call(
        _mm1_body,
        grid=(5,),
        in_specs=[
            pl.BlockSpec((2000, D_IN), lambda i: (i, 0)),
            pl.BlockSpec((D_IN, D_HID), lambda i: (0, 0)),
        ],
        out_specs=pl.BlockSpec((2000, D_HID), lambda i: (i, 0)),
        out_shape=jax.ShapeDtypeStruct((NPAD, D_HID), jnp.float32),
    )(x, W1)

    z = _sc_agg(h0, srcp, dstp, b1)

    out = pl.pallas_call(
        _head_body,
        grid=(5,),
        in_specs=[
            pl.BlockSpec((2000, D_HID), lambda i: (i, 0)),
            pl.BlockSpec((D_HID, N_CLASSES), lambda i: (0, 0)),
            pl.BlockSpec((1, N_CLASSES), lambda i: (0, 0)),
        ],
        out_specs=pl.BlockSpec((2000, N_CLASSES), lambda i: (i, 0)),
        out_shape=jax.ShapeDtypeStruct((N, N_CLASSES), jnp.float32),
    )(z, W2, b2.reshape(1, N_CLASSES))
    return out
